# baseline (speedup 1.0000x reference)
import jax
import jax.numpy as jnp
from jax import lax
from jax.experimental import pallas as pl
from jax.experimental.pallas import tpu as pltpu
from jax.experimental.pallas import tpu_sc as plsc

N = 10000
NP = 10240
E = 160000
D = 256
DH = D // 2
NSC = 16
EPS = E // NSC
K = 80
NB = EPS // K
NW = 25
RPS = NP // NSC
RB = 2000


def _stage1_body(x_ref, w_ref, b_ref, c_ref, t0_ref, t1_ref):
    x = x_ref[...]
    c = c_ref[0, 0]
    sq = jnp.sqrt(c)
    nrm = jnp.sqrt(jnp.sum(x * x, axis=1, keepdims=True))
    z = sq * nrm
    atz = 0.5 * jnp.log((1.0 + z) / (1.0 - z))
    tang = x * (2.0 / sq * atz / nrm)
    res = lax.dot_general(tang, w_ref[...], (((1,), (1,)), ((), ())),
                          preferred_element_type=jnp.float32) + b_ref[...]
    t0_ref[...] = res[:, :DH]
    t1_ref[...] = res[:, DH:]


def _transform(x, w, b2, c2):
    return pl.pallas_call(
        _stage1_body,
        grid=(N // RB,),
        in_specs=[
            pl.BlockSpec((RB, D), lambda i: (i, 0)),
            pl.BlockSpec((D, D), lambda i: (0, 0)),
            pl.BlockSpec((1, D), lambda i: (0, 0)),
            pl.BlockSpec(memory_space=pltpu.SMEM),
        ],
        out_specs=[
            pl.BlockSpec((RB, DH), lambda i: (i, 0)),
            pl.BlockSpec((RB, DH), lambda i: (i, 0)),
        ],
        out_shape=[
            jax.ShapeDtypeStruct((N, DH), jnp.float32),
            jax.ShapeDtypeStruct((N, DH), jnp.float32),
        ],
    )(x, w, b2, c2)


def _sc_body(t0_hbm, t1_hbm, ei_hbm, s0_hbm, s1_hbm, cnt_hbm,
             acc, cacc, sidx, didx, r0, r1, r2, ones,
             sem0, sem1, sem2, semc, sems_s):
    cid = lax.axis_index("c")
    sid = lax.axis_index("s")
    base = sid * RPS

    def fz(i, _):
        for j in range(DH // 16):
            r0[i, pl.ds(j * 16, 16)] = jnp.zeros((16,), jnp.float32)
        ones[i] = jnp.zeros((16,), jnp.float32)
        return 0
    lax.fori_loop(0, K, fz, 0)

    for k in range(RPS // K):
        pltpu.sync_copy(r0, acc.at[pl.ds(base + k * K, K)])

    @pl.when(cid == 0)
    def _():
        for k in range(RPS // K):
            pltpu.sync_copy(ones, cacc.at[pl.ds(base + k * K, K)])

    def fo(i, _):
        ones[i] = jnp.ones((16,), jnp.float32)
        return 0
    lax.fori_loop(0, K, fo, 0)

    plsc.subcore_barrier()

    bufs = (r0, r1, r2)
    sems = (sem0, sem1, sem2)

    def do_edges(t_hbm, with_cnt):
        def gather(i, r):
            pltpu.async_copy(t_hbm.at[sidx.at[i]], bufs[r], sems[r])

        def gwait(r):
            pltpu.make_async_copy(t_hbm.at[pl.ds(0, K)], bufs[r], sems[r]).wait()

        def scatter(i, r):
            pltpu.async_copy(bufs[r], acc.at[didx.at[i]], sems_s, add=True)
            if with_cnt:
                pltpu.async_copy(ones, cacc.at[didx.at[i]], semc, add=True)

        def swait():
            pltpu.make_async_copy(r0, acc.at[pl.ds(0, K)], sems_s).wait()

        def window(w, _):
            pltpu.sync_copy(ei_hbm.at[0, sid, pl.ds(w * NW, NW)], sidx)
            pltpu.sync_copy(ei_hbm.at[1, sid, pl.ds(w * NW, NW)], didx)
            gather(0, 0)
            gather(1, 1)

            def body(j, _):
                i0 = 3 * j
                for r in range(3):
                    i = i0 + r
                    gwait(r)

                    @pl.when(i > 0)
                    def _():
                        swait()

                    @pl.when(i + 2 < NW)
                    def _():
                        gather(i + 2, (r + 2) % 3)
                    scatter(i, r)
                return 0
            lax.fori_loop(0, NW // 3, body, 0)
            gwait(0)
            swait()
            scatter(NW - 1, 0)
            swait()
            if with_cnt:
                def drain(_, __):
                    pltpu.make_async_copy(cnt_hbm.at[pl.ds(0, K)],
                                          cacc.at[pl.ds(0, K)], semc).wait()
                    return 0
                lax.fori_loop(0, NW, drain, 0)
            return 0
        lax.fori_loop(0, NB // NW, window, 0)

    @pl.when(cid == 0)
    def _():
        do_edges(t0_hbm, True)
        plsc.subcore_barrier()
        pltpu.sync_copy(acc.at[pl.ds(base, RPS)], s0_hbm.at[pl.ds(base, RPS)])
        pltpu.sync_copy(cacc.at[pl.ds(base, RPS)], cnt_hbm.at[pl.ds(base, RPS)])

    @pl.when(cid == 1)
    def _():
        do_edges(t1_hbm, False)
        plsc.subcore_barrier()
        pltpu.sync_copy(acc.at[pl.ds(base, RPS)], s1_hbm.at[pl.ds(base, RPS)])


def _scatter_mean(t0, t1, ei4):
    mesh = plsc.VectorSubcoreMesh(core_axis_name="c", subcore_axis_name="s")
    f = pl.kernel(
        _sc_body,
        out_type=[
            jax.ShapeDtypeStruct((NP, DH), jnp.float32),
            jax.ShapeDtypeStruct((NP, DH), jnp.float32),
            jax.ShapeDtypeStruct((NP, 16), jnp.float32),
        ],
        mesh=mesh,
        scratch_types=[
            pltpu.VMEM_SHARED((NP, DH), jnp.float32),
            pltpu.VMEM_SHARED((NP, 16), jnp.float32),
            pltpu.VMEM((NW, K), jnp.int32),
            pltpu.VMEM((NW, K), jnp.int32),
            pltpu.VMEM((K, DH), jnp.float32),
            pltpu.VMEM((K, DH), jnp.float32),
            pltpu.VMEM((K, DH), jnp.float32),
            pltpu.VMEM((K, 16), jnp.float32),
            pltpu.SemaphoreType.DMA,
            pltpu.SemaphoreType.DMA,
            pltpu.SemaphoreType.DMA,
            pltpu.SemaphoreType.DMA,
            pltpu.SemaphoreType.DMA,
        ],
        compiler_params=pltpu.CompilerParams(use_tc_tiling_on_sc=False),
    )
    return f(t0, t1, ei4)


def _stage3_body(s0_ref, s1_ref, cnt_ref, c_ref, out_ref):
    s = jnp.concatenate([s0_ref[...], s1_ref[...]], axis=1)
    cntv = cnt_ref[:, 0:1]
    neigh = s / jnp.maximum(cntv, 1.0)
    c = c_ref[0, 0]
    sq = jnp.sqrt(c)
    nv = jnp.sqrt(jnp.sum(neigh * neigh, axis=1, keepdims=True))
    out_ref[...] = jnp.tanh(sq * nv * 0.5) * neigh / (sq * nv)


def _expmap(s0, s1, cnt, c2):
    return pl.pallas_call(
        _stage3_body,
        grid=(N // RB,),
        in_specs=[
            pl.BlockSpec((RB, DH), lambda i: (i, 0)),
            pl.BlockSpec((RB, DH), lambda i: (i, 0)),
            pl.BlockSpec((RB, 16), lambda i: (i, 0)),
            pl.BlockSpec(memory_space=pltpu.SMEM),
        ],
        out_specs=pl.BlockSpec((RB, D), lambda i: (i, 0)),
        out_shape=jax.ShapeDtypeStruct((N, D), jnp.float32),
    )(s0, s1, cnt, c2)


def kernel(node_embeddings, edge_index, lin_w, lin_b, curvature):
    c2 = curvature.reshape(1, 1)
    b2 = lin_b.reshape(1, D)
    t0, t1 = _transform(node_embeddings, lin_w, b2, c2)
    ei4 = edge_index.reshape(2, NSC, NB, K)
    s0, s1, cnt = _scatter_mean(t0, t1, ei4)
    return _expmap(s0, s1, cnt, c2)


# speedup vs baseline: 1.0020x; 1.0020x over previous
import jax
import jax.numpy as jnp
from jax import lax
from jax.experimental import pallas as pl
from jax.experimental.pallas import tpu as pltpu
from jax.experimental.pallas import tpu_sc as plsc

N = 10000
NP = 10240
E = 160000
D = 256
DH = D // 2
NSC = 16
EPS = E // NSC
K = 80
NB = EPS // K
NW = 25
RPS = NP // NSC
RB = 2000


def _stage1_body(x_ref, w_ref, b_ref, c_ref, t0_ref, t1_ref):
    x = x_ref[...]
    c = c_ref[0, 0]
    sq = jnp.sqrt(c)
    nrm = jnp.sqrt(jnp.sum(x * x, axis=1, keepdims=True))
    z = sq * nrm
    atz = 0.5 * jnp.log((1.0 + z) / (1.0 - z))
    tang = x * (2.0 / sq * atz / nrm)
    res = lax.dot_general(tang, w_ref[...], (((1,), (1,)), ((), ())),
                          preferred_element_type=jnp.float32) + b_ref[...]
    t0_ref[...] = res[:, :DH]
    t1_ref[...] = res[:, DH:]


def _transform(x, w, b2, c2):
    return pl.pallas_call(
        _stage1_body,
        grid=(N // RB,),
        in_specs=[
            pl.BlockSpec((RB, D), lambda i: (i, 0)),
            pl.BlockSpec((D, D), lambda i: (0, 0)),
            pl.BlockSpec((1, D), lambda i: (0, 0)),
            pl.BlockSpec(memory_space=pltpu.SMEM),
        ],
        out_specs=[
            pl.BlockSpec((RB, DH), lambda i: (i, 0)),
            pl.BlockSpec((RB, DH), lambda i: (i, 0)),
        ],
        out_shape=[
            jax.ShapeDtypeStruct((N, DH), jnp.float32),
            jax.ShapeDtypeStruct((N, DH), jnp.float32),
        ],
    )(x, w, b2, c2)


def _sc_body(t0_hbm, t1_hbm, ei_hbm, s0_hbm, s1_hbm, cnt_hbm,
             acc, cacc, sidx, didx, r0, r1, r2, ones,
             sem0, sem1, sem2, semc):
    cid = lax.axis_index("c")
    sid = lax.axis_index("s")
    base = sid * RPS

    def fz(i, _):
        for j in range(DH // 16):
            r0[i, pl.ds(j * 16, 16)] = jnp.zeros((16,), jnp.float32)
        ones[i] = jnp.zeros((16,), jnp.float32)
        return 0
    lax.fori_loop(0, K, fz, 0)

    for k in range(RPS // K):
        pltpu.sync_copy(r0, acc.at[pl.ds(base + k * K, K)])

    @pl.when(cid == 0)
    def _():
        for k in range(RPS // K):
            pltpu.sync_copy(ones, cacc.at[pl.ds(base + k * K, K)])

    def fo(i, _):
        ones[i] = jnp.ones((16,), jnp.float32)
        return 0
    lax.fori_loop(0, K, fo, 0)

    plsc.subcore_barrier()

    bufs = (r0, r1, r2)
    sems = (sem0, sem1, sem2)

    def do_edges(t_hbm, with_cnt):
        def gather(i, r):
            pltpu.async_copy(t_hbm.at[sidx.at[i]], bufs[r], sems[r])

        def gwait(r):
            pltpu.make_async_copy(t_hbm.at[pl.ds(0, K)], bufs[r], sems[r]).wait()

        def scatter(i, r):
            pltpu.sync_copy(bufs[r], acc.at[didx.at[i]], add=True)
            if with_cnt:
                pltpu.async_copy(ones, cacc.at[didx.at[i]], semc, add=True)

        def window(w, _):
            pltpu.sync_copy(ei_hbm.at[0, sid, pl.ds(w * NW, NW)], sidx)
            pltpu.sync_copy(ei_hbm.at[1, sid, pl.ds(w * NW, NW)], didx)
            gather(0, 0)
            gather(1, 1)

            def body(j, _):
                i0 = 3 * j
                for r in range(3):
                    i = i0 + r
                    gwait(r)

                    @pl.when(i + 2 < NW)
                    def _():
                        gather(i + 2, (r + 2) % 3)
                    scatter(i, r)
                return 0
            lax.fori_loop(0, NW // 3, body, 0)
            gwait(0)
            scatter(NW - 1, 0)
            if with_cnt:
                def drain(_, __):
                    pltpu.make_async_copy(cnt_hbm.at[pl.ds(0, K)],
                                          cacc.at[pl.ds(0, K)], semc).wait()
                    return 0
                lax.fori_loop(0, NW, drain, 0)
            return 0
        lax.fori_loop(0, NB // NW, window, 0)

    @pl.when(cid == 0)
    def _():
        do_edges(t0_hbm, True)
        plsc.subcore_barrier()
        pltpu.sync_copy(acc.at[pl.ds(base, RPS)], s0_hbm.at[pl.ds(base, RPS)])
        pltpu.sync_copy(cacc.at[pl.ds(base, RPS)], cnt_hbm.at[pl.ds(base, RPS)])

    @pl.when(cid == 1)
    def _():
        do_edges(t1_hbm, False)
        plsc.subcore_barrier()
        pltpu.sync_copy(acc.at[pl.ds(base, RPS)], s1_hbm.at[pl.ds(base, RPS)])


def _scatter_mean(t0, t1, ei4):
    mesh = plsc.VectorSubcoreMesh(core_axis_name="c", subcore_axis_name="s")
    f = pl.kernel(
        _sc_body,
        out_type=[
            jax.ShapeDtypeStruct((NP, DH), jnp.float32),
            jax.ShapeDtypeStruct((NP, DH), jnp.float32),
            jax.ShapeDtypeStruct((NP, 16), jnp.float32),
        ],
        mesh=mesh,
        scratch_types=[
            pltpu.VMEM_SHARED((NP, DH), jnp.float32),
            pltpu.VMEM_SHARED((NP, 16), jnp.float32),
            pltpu.VMEM((NW, K), jnp.int32),
            pltpu.VMEM((NW, K), jnp.int32),
            pltpu.VMEM((K, DH), jnp.float32),
            pltpu.VMEM((K, DH), jnp.float32),
            pltpu.VMEM((K, DH), jnp.float32),
            pltpu.VMEM((K, 16), jnp.float32),
            pltpu.SemaphoreType.DMA,
            pltpu.SemaphoreType.DMA,
            pltpu.SemaphoreType.DMA,
            pltpu.SemaphoreType.DMA,
        ],
        compiler_params=pltpu.CompilerParams(use_tc_tiling_on_sc=False),
    )
    return f(t0, t1, ei4)


def _stage3_body(s0_ref, s1_ref, cnt_ref, c_ref, out_ref):
    s = jnp.concatenate([s0_ref[...], s1_ref[...]], axis=1)
    cntv = cnt_ref[:, 0:1]
    neigh = s / jnp.maximum(cntv, 1.0)
    c = c_ref[0, 0]
    sq = jnp.sqrt(c)
    nv = jnp.sqrt(jnp.sum(neigh * neigh, axis=1, keepdims=True))
    out_ref[...] = jnp.tanh(sq * nv * 0.5) * neigh / (sq * nv)


def _expmap(s0, s1, cnt, c2):
    return pl.pallas_call(
        _stage3_body,
        grid=(N // RB,),
        in_specs=[
            pl.BlockSpec((RB, DH), lambda i: (i, 0)),
            pl.BlockSpec((RB, DH), lambda i: (i, 0)),
            pl.BlockSpec((RB, 16), lambda i: (i, 0)),
            pl.BlockSpec(memory_space=pltpu.SMEM),
        ],
        out_specs=pl.BlockSpec((RB, D), lambda i: (i, 0)),
        out_shape=jax.ShapeDtypeStruct((N, D), jnp.float32),
    )(s0, s1, cnt, c2)


def kernel(node_embeddings, edge_index, lin_w, lin_b, curvature):
    c2 = curvature.reshape(1, 1)
    b2 = lin_b.reshape(1, D)
    t0, t1 = _transform(node_embeddings, lin_w, b2, c2)
    ei4 = edge_index.reshape(2, NSC, NB, K)
    s0, s1, cnt = _scatter_mean(t0, t1, ei4)
    return _expmap(s0, s1, cnt, c2)


# speedup vs baseline: 1.0212x; 1.0192x over previous
import jax
import jax.numpy as jnp
from jax import lax
from jax.experimental import pallas as pl
from jax.experimental.pallas import tpu as pltpu
from jax.experimental.pallas import tpu_sc as plsc

N = 10000
NP = 10240
E = 160000
D = 256
DH = D // 2
NSC = 16
EPS = E // NSC
K = 80
NB = EPS // K
NW = 25
RPS = NP // NSC
RB = 2000


def _stage1_body(x_ref, w_ref, b_ref, c_ref, t0_ref, t1_ref):
    x = x_ref[...]
    c = c_ref[0, 0]
    sq = jnp.sqrt(c)
    nrm = jnp.sqrt(jnp.sum(x * x, axis=1, keepdims=True))
    z = sq * nrm
    atz = 0.5 * jnp.log((1.0 + z) / (1.0 - z))
    tang = x * (2.0 / sq * atz / nrm)
    res = lax.dot_general(tang, w_ref[...], (((1,), (1,)), ((), ())),
                          preferred_element_type=jnp.float32) + b_ref[...]
    t0_ref[...] = res[:, :DH]
    t1_ref[...] = res[:, DH:]


def _transform(x, w, b2, c2):
    return pl.pallas_call(
        _stage1_body,
        grid=(N // RB,),
        in_specs=[
            pl.BlockSpec((RB, D), lambda i: (i, 0)),
            pl.BlockSpec((D, D), lambda i: (0, 0)),
            pl.BlockSpec((1, D), lambda i: (0, 0)),
            pl.BlockSpec(memory_space=pltpu.SMEM),
        ],
        out_specs=[
            pl.BlockSpec((RB, DH), lambda i: (i, 0)),
            pl.BlockSpec((RB, DH), lambda i: (i, 0)),
        ],
        out_shape=[
            jax.ShapeDtypeStruct((N, DH), jnp.float32),
            jax.ShapeDtypeStruct((N, DH), jnp.float32),
        ],
    )(x, w, b2, c2)


def _sc_body(t0_hbm, t1_hbm, ei_hbm, s0_hbm, s1_hbm, cnt_hbm,
             acc, cacc, sidx, didx, r0, r1, r2, ones,
             sem0, sem1, sem2, semc, semi):
    cid = lax.axis_index("c")
    sid = lax.axis_index("s")
    base = sid * RPS

    def fz(i, _):
        for j in range(DH // 16):
            r0[i, pl.ds(j * 16, 16)] = jnp.zeros((16,), jnp.float32)
        ones[i] = jnp.zeros((16,), jnp.float32)
        return 0
    lax.fori_loop(0, K, fz, 0)

    for k in range(RPS // K):
        pltpu.sync_copy(r0, acc.at[pl.ds(base + k * K, K)])

    @pl.when(cid == 0)
    def _():
        for k in range(RPS // K):
            pltpu.sync_copy(ones, cacc.at[pl.ds(base + k * K, K)])

    def fo(i, _):
        ones[i] = jnp.ones((16,), jnp.float32)
        return 0
    lax.fori_loop(0, K, fo, 0)

    plsc.subcore_barrier()

    bufs = (r0, r1, r2)
    sems = (sem0, sem1, sem2)

    def do_edges(t_hbm, with_cnt):
        def gather(i, r):
            pltpu.async_copy(t_hbm.at[sidx.at[i]], bufs[r], sems[r])

        def gwait(r):
            pltpu.make_async_copy(t_hbm.at[pl.ds(0, K)], bufs[r], sems[r]).wait()

        def scatter(i, r):
            pltpu.sync_copy(bufs[r], acc.at[didx.at[i]], add=True)
            if with_cnt:
                pltpu.async_copy(ones, cacc.at[didx.at[i]], semc, add=True)

        def iwait():
            pltpu.make_async_copy(ei_hbm.at[0, sid, pl.ds(0, NW)], sidx,
                                  semi).wait()

        def window(w, _):
            @pl.when(w == 0)
            def _():
                pltpu.sync_copy(ei_hbm.at[0, sid, pl.ds(w * NW, NW)], sidx)
                pltpu.sync_copy(ei_hbm.at[1, sid, pl.ds(w * NW, NW)], didx)

            @pl.when(w > 0)
            def _():
                iwait()
                iwait()
            gather(0, 0)
            gather(1, 1)

            def body(j, _):
                i0 = 3 * j
                for r in range(3):
                    i = i0 + r
                    gwait(r)

                    @pl.when(i + 2 < NW)
                    def _():
                        gather(i + 2, (r + 2) % 3)
                    scatter(i, r)
                return 0
            lax.fori_loop(0, NW // 3, body, 0)
            gwait(0)

            @pl.when(w + 1 < NB // NW)
            def _():
                pltpu.async_copy(ei_hbm.at[0, sid, pl.ds((w + 1) * NW, NW)],
                                 sidx, semi)
            scatter(NW - 1, 0)
            if with_cnt:
                def drain(_, __):
                    pltpu.make_async_copy(cnt_hbm.at[pl.ds(0, K)],
                                          cacc.at[pl.ds(0, K)], semc).wait()
                    return 0
                lax.fori_loop(0, NW, drain, 0)

            @pl.when(w + 1 < NB // NW)
            def _():
                pltpu.async_copy(ei_hbm.at[1, sid, pl.ds((w + 1) * NW, NW)],
                                 didx, semi)
            return 0
        lax.fori_loop(0, NB // NW, window, 0)

    @pl.when(cid == 0)
    def _():
        do_edges(t0_hbm, True)
        plsc.subcore_barrier()
        pltpu.sync_copy(acc.at[pl.ds(base, RPS)], s0_hbm.at[pl.ds(base, RPS)])
        pltpu.sync_copy(cacc.at[pl.ds(base, RPS)], cnt_hbm.at[pl.ds(base, RPS)])

    @pl.when(cid == 1)
    def _():
        do_edges(t1_hbm, False)
        plsc.subcore_barrier()
        pltpu.sync_copy(acc.at[pl.ds(base, RPS)], s1_hbm.at[pl.ds(base, RPS)])


def _scatter_mean(t0, t1, ei4):
    mesh = plsc.VectorSubcoreMesh(core_axis_name="c", subcore_axis_name="s")
    f = pl.kernel(
        _sc_body,
        out_type=[
            jax.ShapeDtypeStruct((NP, DH), jnp.float32),
            jax.ShapeDtypeStruct((NP, DH), jnp.float32),
            jax.ShapeDtypeStruct((NP, 16), jnp.float32),
        ],
        mesh=mesh,
        scratch_types=[
            pltpu.VMEM_SHARED((NP, DH), jnp.float32),
            pltpu.VMEM_SHARED((NP, 16), jnp.float32),
            pltpu.VMEM((NW, K), jnp.int32),
            pltpu.VMEM((NW, K), jnp.int32),
            pltpu.VMEM((K, DH), jnp.float32),
            pltpu.VMEM((K, DH), jnp.float32),
            pltpu.VMEM((K, DH), jnp.float32),
            pltpu.VMEM((K, 16), jnp.float32),
            pltpu.SemaphoreType.DMA,
            pltpu.SemaphoreType.DMA,
            pltpu.SemaphoreType.DMA,
            pltpu.SemaphoreType.DMA,
            pltpu.SemaphoreType.DMA,
        ],
        compiler_params=pltpu.CompilerParams(use_tc_tiling_on_sc=False),
    )
    return f(t0, t1, ei4)


def _stage3_body(s0_ref, s1_ref, cnt_ref, c_ref, out_ref):
    s = jnp.concatenate([s0_ref[...], s1_ref[...]], axis=1)
    cntv = cnt_ref[:, 0:1]
    neigh = s / jnp.maximum(cntv, 1.0)
    c = c_ref[0, 0]
    sq = jnp.sqrt(c)
    nv = jnp.sqrt(jnp.sum(neigh * neigh, axis=1, keepdims=True))
    out_ref[...] = jnp.tanh(sq * nv * 0.5) * neigh / (sq * nv)


def _expmap(s0, s1, cnt, c2):
    return pl.pallas_call(
        _stage3_body,
        grid=(N // RB,),
        in_specs=[
            pl.BlockSpec((RB, DH), lambda i: (i, 0)),
            pl.BlockSpec((RB, DH), lambda i: (i, 0)),
            pl.BlockSpec((RB, 16), lambda i: (i, 0)),
            pl.BlockSpec(memory_space=pltpu.SMEM),
        ],
        out_specs=pl.BlockSpec((RB, D), lambda i: (i, 0)),
        out_shape=jax.ShapeDtypeStruct((N, D), jnp.float32),
    )(s0, s1, cnt, c2)


def kernel(node_embeddings, edge_index, lin_w, lin_b, curvature):
    c2 = curvature.reshape(1, 1)
    b2 = lin_b.reshape(1, D)
    t0, t1 = _transform(node_embeddings, lin_w, b2, c2)
    ei4 = edge_index.reshape(2, NSC, NB, K)
    s0, s1, cnt = _scatter_mean(t0, t1, ei4)
    return _expmap(s0, s1, cnt, c2)
